# baseline (speedup 1.0000x reference)
import functools

import jax
import jax.numpy as jnp
from jax import lax
from jax.experimental import pallas as pl
from jax.experimental.pallas import tpu as pltpu
from jax.experimental.pallas import tpu_sc as plsc

_NC, _NS = 2, 16
_NW = _NC * _NS
_D = 1024
_BATCH = 1024
_SEQ = 50
_B_PER_W = _BATCH // _NW


def _make_gather():
    mesh = plsc.VectorSubcoreMesh(core_axis_name="c", subcore_axis_name="s",
                                  num_cores=2)

    @functools.partial(
        pl.kernel,
        mesh=mesh,
        out_type=jax.ShapeDtypeStruct((_BATCH, _SEQ, _D), jnp.float32),
        scratch_types=[
            pltpu.VMEM((_B_PER_W, 56), jnp.int32),
            pltpu.VMEM((56, _D), jnp.float32),
            pltpu.VMEM((56, _D), jnp.float32),
            pltpu.SemaphoreType.DMA,
            pltpu.SemaphoreType.DMA,
            pltpu.SemaphoreType.DMA,
            pltpu.SemaphoreType.DMA,
        ],
    )
    def gather_rows(table_hbm, idx_hbm, out_hbm,
                    idx_v, buf0, buf1, g0, g1, o0, o1):
        wid = lax.axis_index("s") * _NC + lax.axis_index("c")
        base = wid * _B_PER_W
        pltpu.sync_copy(idx_hbm.at[pl.ds(base, _B_PER_W)], idx_v)

        def gather(g, buf, sem):
            return pltpu.make_async_copy(
                table_hbm.at[idx_v.at[g]], buf, sem)

        def out_copy(g, buf, sem):
            return pltpu.make_async_copy(
                buf, out_hbm.at[base + g].at[pl.ds(0, 56)], sem)

        gather(0, buf0, g0).start()

        def body(p, carry):
            ga = 2 * p
            gather(ga, buf0, g0).wait()
            out_copy(ga, buf0, o0).start()
            gather(ga + 1, buf1, g1).start()
            gather(ga + 1, buf1, g1).wait()
            out_copy(ga + 1, buf1, o1).start()
            out_copy(ga, buf0, o0).wait()

            @pl.when(p < _B_PER_W // 2 - 1)
            def _():
                gather(ga + 2, buf0, g0).start()

            out_copy(ga + 1, buf1, o1).wait()
            return carry

        lax.fori_loop(0, _B_PER_W // 2, body, 0)

    return gather_rows


_gather = _make_gather()


def kernel(indices, table):
    idx56 = jnp.concatenate([indices, indices[:, :6]], axis=1)
    return _gather(table, idx56)


# speedup vs baseline: 1.0002x; 1.0002x over previous
import functools

import jax
import jax.numpy as jnp
from jax import lax
from jax.experimental import pallas as pl
from jax.experimental.pallas import tpu as pltpu
from jax.experimental.pallas import tpu_sc as plsc

_NC, _NS = 2, 16
_NW = _NC * _NS
_D = 1024
_BATCH = 1024
_SEQ = 50
_B_PER_W = _BATCH // _NW


def _make_gather():
    mesh = plsc.VectorSubcoreMesh(core_axis_name="c", subcore_axis_name="s",
                                  num_cores=2)

    @functools.partial(
        pl.kernel,
        mesh=mesh,
        compiler_params=pltpu.CompilerParams(skip_device_barrier=True),
        out_type=jax.ShapeDtypeStruct((_BATCH, _SEQ, _D), jnp.float32),
        scratch_types=[
            pltpu.VMEM((_B_PER_W, 56), jnp.int32),
            pltpu.VMEM((56, _D), jnp.float32),
            pltpu.VMEM((56, _D), jnp.float32),
            pltpu.SemaphoreType.DMA,
            pltpu.SemaphoreType.DMA,
            pltpu.SemaphoreType.DMA,
            pltpu.SemaphoreType.DMA,
        ],
    )
    def gather_rows(table_hbm, idx_hbm, out_hbm,
                    idx_v, buf0, buf1, g0, g1, o0, o1):
        wid = lax.axis_index("s") * _NC + lax.axis_index("c")
        base = wid * _B_PER_W
        pltpu.sync_copy(idx_hbm.at[pl.ds(base, _B_PER_W)], idx_v)

        def gather(g, buf, sem):
            return pltpu.make_async_copy(
                table_hbm.at[idx_v.at[g]], buf, sem)

        def out_copy(g, buf, sem):
            return pltpu.make_async_copy(
                buf, out_hbm.at[base + g].at[pl.ds(0, 56)], sem)

        gather(0, buf0, g0).start()

        def body(p, carry):
            ga = 2 * p
            gather(ga, buf0, g0).wait()
            out_copy(ga, buf0, o0).start()
            gather(ga + 1, buf1, g1).start()
            gather(ga + 1, buf1, g1).wait()
            out_copy(ga + 1, buf1, o1).start()
            out_copy(ga, buf0, o0).wait()

            @pl.when(p < _B_PER_W // 2 - 1)
            def _():
                gather(ga + 2, buf0, g0).start()

            out_copy(ga + 1, buf1, o1).wait()
            return carry

        lax.fori_loop(0, _B_PER_W // 2, body, 0)

    return gather_rows


_gather = _make_gather()


def kernel(indices, table):
    idx56 = jnp.concatenate([indices, indices[:, :6]], axis=1)
    return _gather(table, idx56)


# speedup vs baseline: 2.0831x; 2.0828x over previous
import functools

import jax
import jax.numpy as jnp
from jax import lax
from jax.experimental import pallas as pl
from jax.experimental.pallas import tpu as pltpu
from jax.experimental.pallas import tpu_sc as plsc

_NC, _NS = 2, 16
_NW = _NC * _NS
_D = 1024
_BATCH = 1024
_SEQ = 50
_BPW = _BATCH // _NW


def _make_gather():
    mesh = plsc.VectorSubcoreMesh(core_axis_name="c", subcore_axis_name="s",
                                  num_cores=2)

    @functools.partial(
        pl.kernel,
        mesh=mesh,
        out_type=jax.ShapeDtypeStruct((_SEQ, _BATCH, _D), jnp.float32),
        scratch_types=[
            pltpu.VMEM((_SEQ, _BPW), jnp.int32),
            pltpu.VMEM((_BPW, _D), jnp.float32),
            pltpu.VMEM((_BPW, _D), jnp.float32),
            pltpu.SemaphoreType.DMA,
            pltpu.SemaphoreType.DMA,
            pltpu.SemaphoreType.DMA,
            pltpu.SemaphoreType.DMA,
        ],
    )
    def gather_rows(table_hbm, idx_hbm, out_hbm,
                    idx_v, buf0, buf1, g0, g1, o0, o1):
        wid = lax.axis_index("s") * _NC + lax.axis_index("c")
        base = wid * _BPW
        pltpu.sync_copy(idx_hbm.at[wid], idx_v)

        def gather(s, buf, sem):
            return pltpu.make_async_copy(
                table_hbm.at[idx_v.at[s]], buf, sem)

        def out_copy(s, buf, sem):
            return pltpu.make_async_copy(
                buf, out_hbm.at[s].at[pl.ds(base, _BPW)], sem)

        gather(0, buf0, g0).start()

        def body(p, carry):
            sa = 2 * p
            gather(sa, buf0, g0).wait()
            out_copy(sa, buf0, o0).start()
            gather(sa + 1, buf1, g1).start()
            gather(sa + 1, buf1, g1).wait()
            out_copy(sa + 1, buf1, o1).start()
            out_copy(sa, buf0, o0).wait()

            @pl.when(p < _SEQ // 2 - 1)
            def _():
                gather(sa + 2, buf0, g0).start()

            out_copy(sa + 1, buf1, o1).wait()
            return carry

        lax.fori_loop(0, _SEQ // 2, body, 0)

    return gather_rows


_gather = _make_gather()


def kernel(indices, table):
    idx_a = indices.reshape(_NW, _BPW, _SEQ).transpose(0, 2, 1)
    out_sc = _gather(table, idx_a)
    return out_sc.transpose(1, 0, 2)
